# baseline (speedup 1.0000x reference)
import functools

import jax
import jax.numpy as jnp
from jax import lax
from jax.experimental import pallas as pl
from jax.experimental.pallas import tpu as pltpu
from jax.experimental.pallas import tpu_sc as plsc

N = 10000
H = 128
E = 320000
LAYERS = 3

NC = 2
NS = 16
NW = NC * NS
CH = 80
NCHUNK = 128
EPT = NCHUNK * CH
E_PAD = NW * EPT
NBUF = 4
NPHASE = 4
HALF = NCHUNK // NPHASE
NGROUP = HALF // NBUF
N_ACC = N + 8
RPT = 624
TAIL_OFF = NS * RPT
ZTAIL = N_ACC - TAIL_OFF
OTAIL = N - TAIL_OFF

_SC_MESH = plsc.VectorSubcoreMesh(core_axis_name="c", subcore_axis_name="s")


@functools.partial(
    pl.kernel,
    mesh=_SC_MESH,
    out_type=jax.ShapeDtypeStruct((NC, N, H), jnp.float32),
    scratch_types=[
        pltpu.VMEM((HALF, CH), jnp.int32),
        pltpu.VMEM((HALF, CH), jnp.int32),
        [pltpu.VMEM((CH, H), jnp.float32)] * NBUF,
        pltpu.VMEM_SHARED((N_ACC, H), jnp.float32),
        [pltpu.SemaphoreType.DMA] * NBUF,
        [pltpu.SemaphoreType.DMA] * NBUF,
    ],
)
def _sc_scatter(m_hbm, src_hbm, dst_hbm, zeros_hbm, out_hbm,
                src_v, dst_v, rows, agg_sh, sg, ss):
    c = lax.axis_index("c")
    s = lax.axis_index("s")
    wid = c * NS + s
    pltpu.sync_copy(src_hbm.at[wid, 0], src_v)
    pltpu.sync_copy(dst_hbm.at[wid, 0], dst_v)
    for b in range(NBUF):
        pltpu.async_copy(m_hbm.at[src_v.at[b]], rows[b], sg[b])
    pltpu.sync_copy(zeros_hbm.at[pl.ds(s * RPT, RPT)],
                    agg_sh.at[pl.ds(s * RPT, RPT)])

    @pl.when(s == NS - 1)
    def _zero_tail():
        pltpu.sync_copy(zeros_hbm.at[pl.ds(TAIL_OFF, ZTAIL)],
                        agg_sh.at[pl.ds(TAIL_OFF, ZTAIL)])
    plsc.subcore_barrier()

    for ph in range(NPHASE):
        if ph > 0:
            pltpu.sync_copy(src_hbm.at[wid, ph], src_v)
            pltpu.sync_copy(dst_hbm.at[wid, ph], dst_v)
            for b in range(NBUF):
                pltpu.async_copy(m_hbm.at[src_v.at[b]], rows[b], sg[b])

        def group(g, carry):
            base = g * NBUF
            for b in range(NBUF):
                j = base + b
                pltpu.make_async_copy(m_hbm.at[src_v.at[j]], rows[b],
                                      sg[b]).wait()
                pltpu.async_copy(rows[b], agg_sh.at[dst_v.at[j]], ss[b],
                                 add=True)

            @pl.when(g < NGROUP - 1)
            def _prefetch():
                for b in range(NBUF):
                    j = base + b
                    pltpu.make_async_copy(rows[b], agg_sh.at[dst_v.at[j]],
                                          ss[b]).wait()
                    pltpu.async_copy(m_hbm.at[src_v.at[j + NBUF]], rows[b],
                                     sg[b])
            return carry

        lax.fori_loop(0, NGROUP, group, 0)
        for b in range(NBUF):
            j = (NGROUP - 1) * NBUF + b
            pltpu.make_async_copy(rows[b], agg_sh.at[dst_v.at[j]],
                                  ss[b]).wait()
    plsc.subcore_barrier()
    pltpu.sync_copy(agg_sh.at[pl.ds(s * RPT, RPT)],
                    out_hbm.at[c, pl.ds(s * RPT, RPT)])

    @pl.when(s == NS - 1)
    def _out_tail():
        pltpu.sync_copy(agg_sh.at[pl.ds(TAIL_OFF, OTAIL)],
                        out_hbm.at[c, pl.ds(TAIL_OFF, OTAIL)])


_RB = 1000
_GRID = N // _RB


def _mm_body(x_ref, w_ref, o_ref):
    o_ref[...] = jnp.dot(x_ref[...], w_ref[...],
                         preferred_element_type=jnp.float32)


_mm = pl.pallas_call(
    _mm_body,
    grid=(_GRID,),
    in_specs=[
        pl.BlockSpec((_RB, H), lambda i: (i, 0)),
        pl.BlockSpec((H, H), lambda i: (0, 0)),
    ],
    out_specs=pl.BlockSpec((_RB, H), lambda i: (i, 0)),
    out_shape=jax.ShapeDtypeStruct((N, H), jnp.float32),
)


def _gh_body(x_ref, whh_ref, bhh_ref, o_ref):
    o_ref[...] = jnp.dot(x_ref[...], whh_ref[...],
                         preferred_element_type=jnp.float32) + bhh_ref[...]


_gh = pl.pallas_call(
    _gh_body,
    grid=(_GRID,),
    in_specs=[
        pl.BlockSpec((_RB, H), lambda i: (i, 0)),
        pl.BlockSpec((H, 3 * H), lambda i: (0, 0)),
        pl.BlockSpec((1, 3 * H), lambda i: (0, 0)),
    ],
    out_specs=pl.BlockSpec((_RB, 3 * H), lambda i: (i, 0)),
    out_shape=jax.ShapeDtypeStruct((N, 3 * H), jnp.float32),
)


def _gru_math(parts_ref, x_ref, gh_ref, wih_ref, bih_ref):
    agg = parts_ref[0] + parts_ref[1]
    x = x_ref[...]
    gh = gh_ref[...]
    gi = jnp.dot(agg, wih_ref[...], preferred_element_type=jnp.float32) \
        + bih_ref[...]
    r = jax.nn.sigmoid(gi[:, :H] + gh[:, :H])
    zg = jax.nn.sigmoid(gi[:, H:2 * H] + gh[:, H:2 * H])
    n = jnp.tanh(gi[:, 2 * H:] + r * gh[:, 2 * H:])
    return (1.0 - zg) * n + zg * x


def _gru_body(parts_ref, x_ref, gh_ref, wn_ref, wih_ref, bih_ref,
              xo_ref, mo_ref):
    xn = _gru_math(parts_ref, x_ref, gh_ref, wih_ref, bih_ref)
    xo_ref[...] = xn
    mo_ref[...] = jnp.dot(xn, wn_ref[...], preferred_element_type=jnp.float32)


def _gru_last_body(parts_ref, x_ref, gh_ref, wih_ref, bih_ref, xo_ref):
    xo_ref[...] = _gru_math(parts_ref, x_ref, gh_ref, wih_ref, bih_ref)


_GRU_IN_SPECS = [
    pl.BlockSpec((NC, _RB, H), lambda i: (0, i, 0)),
    pl.BlockSpec((_RB, H), lambda i: (i, 0)),
    pl.BlockSpec((_RB, 3 * H), lambda i: (i, 0)),
]
_W_SPECS = [
    pl.BlockSpec((H, 3 * H), lambda i: (0, 0)),
    pl.BlockSpec((1, 3 * H), lambda i: (0, 0)),
]

_gru = pl.pallas_call(
    _gru_body,
    grid=(_GRID,),
    in_specs=_GRU_IN_SPECS + [pl.BlockSpec((H, H), lambda i: (0, 0))]
    + _W_SPECS,
    out_specs=[
        pl.BlockSpec((_RB, H), lambda i: (i, 0)),
        pl.BlockSpec((_RB, H), lambda i: (i, 0)),
    ],
    out_shape=[
        jax.ShapeDtypeStruct((N, H), jnp.float32),
        jax.ShapeDtypeStruct((N, H), jnp.float32),
    ],
)

_gru_last = pl.pallas_call(
    _gru_last_body,
    grid=(_GRID,),
    in_specs=_GRU_IN_SPECS + _W_SPECS,
    out_specs=pl.BlockSpec((_RB, H), lambda i: (i, 0)),
    out_shape=jax.ShapeDtypeStruct((N, H), jnp.float32),
)


def kernel(z, edge_index, weight, W_ih, W_hh, b_ih, b_hh):
    pad = E_PAD - E
    pad_src = (jnp.arange(pad, dtype=jnp.int32) * 127) % N
    pad_dst = N + (jnp.arange(pad, dtype=jnp.int32) % (N_ACC - N))
    src = jnp.concatenate(
        [edge_index[0].astype(jnp.int32), pad_src]).reshape(
            NW, NPHASE, HALF, CH)
    dst = jnp.concatenate(
        [edge_index[1].astype(jnp.int32), pad_dst]).reshape(
            NW, NPHASE, HALF, CH)
    W_ihT = W_ih.T.astype(jnp.float32)
    W_hhT = W_hh.T.astype(jnp.float32)
    b_ih2 = b_ih.reshape(1, 3 * H)
    b_hh2 = b_hh.reshape(1, 3 * H)
    zeros = jnp.zeros((N_ACC, H), jnp.float32)

    x = z
    m = _mm(x, weight[0])
    for i in range(LAYERS):
        parts = _sc_scatter(m, src, dst, zeros)
        gh = _gh(x, W_hhT, b_hh2)
        if i < LAYERS - 1:
            x, m = _gru(parts, x, gh, weight[i + 1], W_ihT, b_ih2)
        else:
            x = _gru_last(parts, x, gh, W_ihT, b_ih2)
    return x


# speedup vs baseline: 1.0322x; 1.0322x over previous
import functools

import jax
import jax.numpy as jnp
from jax import lax
from jax.experimental import pallas as pl
from jax.experimental.pallas import tpu as pltpu
from jax.experimental.pallas import tpu_sc as plsc

N = 10000
H = 128
E = 320000
LAYERS = 3

NC = 2
NS = 16
NW = NC * NS
CH = 80
NCHUNK = 128
EPT = NCHUNK * CH
E_PAD = NW * EPT
NBUF = 4
NPHASE = 4
HALF = NCHUNK // NPHASE
NGROUP = HALF // NBUF
N_ACC = N + 8
RPT = 624
TAIL_OFF = NS * RPT
ZTAIL = N_ACC - TAIL_OFF
OTAIL = N - TAIL_OFF

_SC_MESH = plsc.VectorSubcoreMesh(core_axis_name="c", subcore_axis_name="s")


@functools.partial(
    pl.kernel,
    mesh=_SC_MESH,
    out_type=jax.ShapeDtypeStruct((NC, N, H), jnp.float32),
    scratch_types=[
        pltpu.VMEM((HALF, CH), jnp.int32),
        pltpu.VMEM((HALF, CH), jnp.int32),
        [pltpu.VMEM((CH, H), jnp.float32)] * NBUF,
        pltpu.VMEM_SHARED((N_ACC, H), jnp.float32),
        [pltpu.SemaphoreType.DMA] * NBUF,
        [pltpu.SemaphoreType.DMA] * NBUF,
    ],
)
def _sc_scatter(m_hbm, src_hbm, dst_hbm, zeros_hbm, out_hbm,
                src_v, dst_v, rows, agg_sh, sg, ss):
    c = lax.axis_index("c")
    s = lax.axis_index("s")
    wid = c * NS + s
    pltpu.sync_copy(src_hbm.at[wid, 0], src_v)
    pltpu.sync_copy(dst_hbm.at[wid, 0], dst_v)
    for b in range(NBUF):
        pltpu.async_copy(m_hbm.at[src_v.at[b]], rows[b], sg[b])
    pltpu.sync_copy(zeros_hbm.at[pl.ds(s * RPT, RPT)],
                    agg_sh.at[pl.ds(s * RPT, RPT)])

    @pl.when(s == NS - 1)
    def _zero_tail():
        pltpu.sync_copy(zeros_hbm.at[pl.ds(TAIL_OFF, ZTAIL)],
                        agg_sh.at[pl.ds(TAIL_OFF, ZTAIL)])
    plsc.subcore_barrier()

    for ph in range(NPHASE):
        if ph > 0:
            pltpu.sync_copy(src_hbm.at[wid, ph], src_v)
            pltpu.sync_copy(dst_hbm.at[wid, ph], dst_v)
            for b in range(NBUF):
                pltpu.async_copy(m_hbm.at[src_v.at[b]], rows[b], sg[b])

        def group(g, carry):
            base = g * NBUF
            for b in range(NBUF):
                j = base + b
                pltpu.make_async_copy(m_hbm.at[src_v.at[j]], rows[b],
                                      sg[b]).wait()
                pltpu.async_copy(rows[b], agg_sh.at[dst_v.at[j]], ss[b],
                                 add=True)

            @pl.when(g < NGROUP - 1)
            def _prefetch():
                for b in range(NBUF):
                    j = base + b
                    pltpu.make_async_copy(rows[b], agg_sh.at[dst_v.at[j]],
                                          ss[b]).wait()
                    pltpu.async_copy(m_hbm.at[src_v.at[j + NBUF]], rows[b],
                                     sg[b])
            return carry

        lax.fori_loop(0, NGROUP, group, 0)
        for b in range(NBUF):
            j = (NGROUP - 1) * NBUF + b
            pltpu.make_async_copy(rows[b], agg_sh.at[dst_v.at[j]],
                                  ss[b]).wait()
    plsc.subcore_barrier()
    pltpu.sync_copy(agg_sh.at[pl.ds(s * RPT, RPT)],
                    out_hbm.at[c, pl.ds(s * RPT, RPT)])

    @pl.when(s == NS - 1)
    def _out_tail():
        pltpu.sync_copy(agg_sh.at[pl.ds(TAIL_OFF, OTAIL)],
                        out_hbm.at[c, pl.ds(TAIL_OFF, OTAIL)])


_RB = 1000
_GRID = N // _RB


def _mm_body(x_ref, w_ref, o_ref):
    o_ref[...] = jnp.dot(x_ref[...], w_ref[...],
                         preferred_element_type=jnp.float32)


_mm = pl.pallas_call(
    _mm_body,
    grid=(_GRID,),
    in_specs=[
        pl.BlockSpec((_RB, H), lambda i: (i, 0)),
        pl.BlockSpec((H, H), lambda i: (0, 0)),
    ],
    out_specs=pl.BlockSpec((_RB, H), lambda i: (i, 0)),
    out_shape=jax.ShapeDtypeStruct((N, H), jnp.float32),
)


def _gru_math(parts_ref, x_ref, wih_ref, whh_ref, bih_ref, bhh_ref):
    agg = parts_ref[0] + parts_ref[1]
    x = x_ref[...]
    gi = jnp.dot(agg, wih_ref[...], preferred_element_type=jnp.float32) \
        + bih_ref[...]
    gh = jnp.dot(x, whh_ref[...], preferred_element_type=jnp.float32) \
        + bhh_ref[...]
    r = jax.nn.sigmoid(gi[:, :H] + gh[:, :H])
    zg = jax.nn.sigmoid(gi[:, H:2 * H] + gh[:, H:2 * H])
    n = jnp.tanh(gi[:, 2 * H:] + r * gh[:, 2 * H:])
    return (1.0 - zg) * n + zg * x


def _gru_body(parts_ref, x_ref, wn_ref, wih_ref, whh_ref, bih_ref, bhh_ref,
              xo_ref, mo_ref):
    xn = _gru_math(parts_ref, x_ref, wih_ref, whh_ref, bih_ref, bhh_ref)
    xo_ref[...] = xn
    mo_ref[...] = jnp.dot(xn, wn_ref[...], preferred_element_type=jnp.float32)


def _gru_last_body(parts_ref, x_ref, wih_ref, whh_ref, bih_ref, bhh_ref,
                   xo_ref):
    xo_ref[...] = _gru_math(parts_ref, x_ref, wih_ref, whh_ref, bih_ref,
                            bhh_ref)


_GRU_IN_SPECS = [
    pl.BlockSpec((NC, _RB, H), lambda i: (0, i, 0)),
    pl.BlockSpec((_RB, H), lambda i: (i, 0)),
]
_W_SPECS = [
    pl.BlockSpec((H, 3 * H), lambda i: (0, 0)),
    pl.BlockSpec((H, 3 * H), lambda i: (0, 0)),
    pl.BlockSpec((1, 3 * H), lambda i: (0, 0)),
    pl.BlockSpec((1, 3 * H), lambda i: (0, 0)),
]

_gru = pl.pallas_call(
    _gru_body,
    grid=(_GRID,),
    in_specs=_GRU_IN_SPECS + [pl.BlockSpec((H, H), lambda i: (0, 0))]
    + _W_SPECS,
    out_specs=[
        pl.BlockSpec((_RB, H), lambda i: (i, 0)),
        pl.BlockSpec((_RB, H), lambda i: (i, 0)),
    ],
    out_shape=[
        jax.ShapeDtypeStruct((N, H), jnp.float32),
        jax.ShapeDtypeStruct((N, H), jnp.float32),
    ],
)

_gru_last = pl.pallas_call(
    _gru_last_body,
    grid=(_GRID,),
    in_specs=_GRU_IN_SPECS + _W_SPECS,
    out_specs=pl.BlockSpec((_RB, H), lambda i: (i, 0)),
    out_shape=jax.ShapeDtypeStruct((N, H), jnp.float32),
)


def kernel(z, edge_index, weight, W_ih, W_hh, b_ih, b_hh):
    pad = E_PAD - E
    pad_src = (jnp.arange(pad, dtype=jnp.int32) * 127) % N
    pad_dst = N + (jnp.arange(pad, dtype=jnp.int32) % (N_ACC - N))
    src = jnp.concatenate(
        [edge_index[0].astype(jnp.int32), pad_src]).reshape(
            NW, NPHASE, HALF, CH)
    dst = jnp.concatenate(
        [edge_index[1].astype(jnp.int32), pad_dst]).reshape(
            NW, NPHASE, HALF, CH)
    W_ihT = W_ih.T.astype(jnp.float32)
    W_hhT = W_hh.T.astype(jnp.float32)
    b_ih2 = b_ih.reshape(1, 3 * H)
    b_hh2 = b_hh.reshape(1, 3 * H)
    zeros = jnp.zeros((N_ACC, H), jnp.float32)

    x = z
    m = _mm(x, weight[0])
    for i in range(LAYERS):
        parts = _sc_scatter(m, src, dst, zeros)
        if i < LAYERS - 1:
            x, m = _gru(parts, x, weight[i + 1], W_ihT, W_hhT, b_ih2, b_hh2)
        else:
            x = _gru_last(parts, x, W_ihT, W_hhT, b_ih2, b_hh2)
    return x


# speedup vs baseline: 1.0531x; 1.0203x over previous
import functools

import jax
import jax.numpy as jnp
from jax import lax
from jax.experimental import pallas as pl
from jax.experimental.pallas import tpu as pltpu
from jax.experimental.pallas import tpu_sc as plsc

N = 10000
H = 128
E = 320000
LAYERS = 3

NC = 2
NS = 16
NW = NC * NS
CH = 80
NCHUNK = 128
EPT = NCHUNK * CH
E_PAD = NW * EPT
NBUF = 4
NPHASE = 4
HALF = NCHUNK // NPHASE
NGROUP = HALF // NBUF
N_ACC = N + 8
RPT = 624
TAIL_OFF = NS * RPT
ZTAIL = N_ACC - TAIL_OFF
OTAIL = N - TAIL_OFF

_SC_MESH = plsc.VectorSubcoreMesh(core_axis_name="c", subcore_axis_name="s")


@functools.partial(
    pl.kernel,
    mesh=_SC_MESH,
    out_type=jax.ShapeDtypeStruct((NC, N, H), jnp.float32),
    scratch_types=[
        pltpu.VMEM((HALF, CH), jnp.int32),
        pltpu.VMEM((HALF, CH), jnp.int32),
        [pltpu.VMEM((CH, H), jnp.float32)] * NBUF,
        pltpu.VMEM_SHARED((N_ACC, H), jnp.float32),
        [pltpu.SemaphoreType.DMA] * NBUF,
        [pltpu.SemaphoreType.DMA] * NBUF,
    ],
)
def _sc_scatter(m_hbm, src_hbm, dst_hbm, zeros_hbm, out_hbm,
                src_v, dst_v, rows, agg_sh, sg, ss):
    c = lax.axis_index("c")
    s = lax.axis_index("s")
    wid = c * NS + s
    pltpu.sync_copy(src_hbm.at[wid, 0], src_v)
    pltpu.sync_copy(dst_hbm.at[wid, 0], dst_v)
    for b in range(NBUF):
        pltpu.async_copy(m_hbm.at[src_v.at[b]], rows[b], sg[b])
    pltpu.sync_copy(zeros_hbm.at[pl.ds(s * RPT, RPT)],
                    agg_sh.at[pl.ds(s * RPT, RPT)])

    @pl.when(s == NS - 1)
    def _zero_tail():
        pltpu.sync_copy(zeros_hbm.at[pl.ds(TAIL_OFF, ZTAIL)],
                        agg_sh.at[pl.ds(TAIL_OFF, ZTAIL)])
    plsc.subcore_barrier()

    for ph in range(NPHASE):
        if ph > 0:
            pltpu.sync_copy(src_hbm.at[wid, ph], src_v)
            pltpu.sync_copy(dst_hbm.at[wid, ph], dst_v)
            for b in range(NBUF):
                pltpu.async_copy(m_hbm.at[src_v.at[b]], rows[b], sg[b])

        def group(g, carry):
            base = g * NBUF
            for b in range(NBUF):
                j = base + b
                pltpu.make_async_copy(m_hbm.at[src_v.at[j]], rows[b],
                                      sg[b]).wait()
                pltpu.async_copy(rows[b], agg_sh.at[dst_v.at[j]], ss[b],
                                 add=True)

            @pl.when(g < NGROUP - 1)
            def _prefetch():
                for b in range(NBUF):
                    j = base + b
                    pltpu.make_async_copy(rows[b], agg_sh.at[dst_v.at[j]],
                                          ss[b]).wait()
                    pltpu.async_copy(m_hbm.at[src_v.at[j + NBUF]], rows[b],
                                     sg[b])
            return carry

        lax.fori_loop(0, NGROUP, group, 0)
        for b in range(NBUF):
            j = (NGROUP - 1) * NBUF + b
            pltpu.make_async_copy(rows[b], agg_sh.at[dst_v.at[j]],
                                  ss[b]).wait()
    plsc.subcore_barrier()
    pltpu.sync_copy(agg_sh.at[pl.ds(s * RPT, RPT)],
                    out_hbm.at[c, pl.ds(s * RPT, RPT)])

    @pl.when(s == NS - 1)
    def _out_tail():
        pltpu.sync_copy(agg_sh.at[pl.ds(TAIL_OFF, OTAIL)],
                        out_hbm.at[c, pl.ds(TAIL_OFF, OTAIL)])


_RB = 2000
_GRID = N // _RB


def _mm_body(x_ref, w_ref, o_ref):
    o_ref[...] = jnp.dot(x_ref[...], w_ref[...],
                         preferred_element_type=jnp.float32)


_mm = pl.pallas_call(
    _mm_body,
    grid=(_GRID,),
    in_specs=[
        pl.BlockSpec((_RB, H), lambda i: (i, 0)),
        pl.BlockSpec((H, H), lambda i: (0, 0)),
    ],
    out_specs=pl.BlockSpec((_RB, H), lambda i: (i, 0)),
    out_shape=jax.ShapeDtypeStruct((N, H), jnp.float32),
)


def _gru_math(parts_ref, x_ref, wih_ref, whh_ref, bih_ref, bhh_ref):
    agg = parts_ref[0] + parts_ref[1]
    x = x_ref[...]
    gi = jnp.dot(agg, wih_ref[...], preferred_element_type=jnp.float32) \
        + bih_ref[...]
    gh = jnp.dot(x, whh_ref[...], preferred_element_type=jnp.float32) \
        + bhh_ref[...]
    r = jax.nn.sigmoid(gi[:, :H] + gh[:, :H])
    zg = jax.nn.sigmoid(gi[:, H:2 * H] + gh[:, H:2 * H])
    n = jnp.tanh(gi[:, 2 * H:] + r * gh[:, 2 * H:])
    return (1.0 - zg) * n + zg * x


def _gru_body(parts_ref, x_ref, wn_ref, wih_ref, whh_ref, bih_ref, bhh_ref,
              xo_ref, mo_ref):
    xn = _gru_math(parts_ref, x_ref, wih_ref, whh_ref, bih_ref, bhh_ref)
    xo_ref[...] = xn
    mo_ref[...] = jnp.dot(xn, wn_ref[...], preferred_element_type=jnp.float32)


def _gru_last_body(parts_ref, x_ref, wih_ref, whh_ref, bih_ref, bhh_ref,
                   xo_ref):
    xo_ref[...] = _gru_math(parts_ref, x_ref, wih_ref, whh_ref, bih_ref,
                            bhh_ref)


_GRU_IN_SPECS = [
    pl.BlockSpec((NC, _RB, H), lambda i: (0, i, 0)),
    pl.BlockSpec((_RB, H), lambda i: (i, 0)),
]
_W_SPECS = [
    pl.BlockSpec((H, 3 * H), lambda i: (0, 0)),
    pl.BlockSpec((H, 3 * H), lambda i: (0, 0)),
    pl.BlockSpec((1, 3 * H), lambda i: (0, 0)),
    pl.BlockSpec((1, 3 * H), lambda i: (0, 0)),
]

_gru = pl.pallas_call(
    _gru_body,
    grid=(_GRID,),
    in_specs=_GRU_IN_SPECS + [pl.BlockSpec((H, H), lambda i: (0, 0))]
    + _W_SPECS,
    out_specs=[
        pl.BlockSpec((_RB, H), lambda i: (i, 0)),
        pl.BlockSpec((_RB, H), lambda i: (i, 0)),
    ],
    out_shape=[
        jax.ShapeDtypeStruct((N, H), jnp.float32),
        jax.ShapeDtypeStruct((N, H), jnp.float32),
    ],
)

_gru_last = pl.pallas_call(
    _gru_last_body,
    grid=(_GRID,),
    in_specs=_GRU_IN_SPECS + _W_SPECS,
    out_specs=pl.BlockSpec((_RB, H), lambda i: (i, 0)),
    out_shape=jax.ShapeDtypeStruct((N, H), jnp.float32),
)


def kernel(z, edge_index, weight, W_ih, W_hh, b_ih, b_hh):
    pad = E_PAD - E
    pad_src = (jnp.arange(pad, dtype=jnp.int32) * 127) % N
    pad_dst = N + (jnp.arange(pad, dtype=jnp.int32) % (N_ACC - N))
    src = jnp.concatenate(
        [edge_index[0].astype(jnp.int32), pad_src]).reshape(
            NW, NPHASE, HALF, CH)
    dst = jnp.concatenate(
        [edge_index[1].astype(jnp.int32), pad_dst]).reshape(
            NW, NPHASE, HALF, CH)
    W_ihT = W_ih.T.astype(jnp.float32)
    W_hhT = W_hh.T.astype(jnp.float32)
    b_ih2 = b_ih.reshape(1, 3 * H)
    b_hh2 = b_hh.reshape(1, 3 * H)
    zeros = jnp.zeros((N_ACC, H), jnp.float32)

    x = z
    m = _mm(x, weight[0])
    for i in range(LAYERS):
        parts = _sc_scatter(m, src, dst, zeros)
        if i < LAYERS - 1:
            x, m = _gru(parts, x, weight[i + 1], W_ihT, W_hhT, b_ih2, b_hh2)
        else:
            x = _gru_last(parts, x, W_ihT, W_hhT, b_ih2, b_hh2)
    return x


# speedup vs baseline: 1.0606x; 1.0071x over previous
import functools

import jax
import jax.numpy as jnp
from jax import lax
from jax.experimental import pallas as pl
from jax.experimental.pallas import tpu as pltpu
from jax.experimental.pallas import tpu_sc as plsc

N = 10000
H = 128
E = 320000
LAYERS = 3

NC = 2
NS = 16
NW = NC * NS
CH = 80
NCHUNK = 128
EPT = NCHUNK * CH
E_PAD = NW * EPT
NBUF = 4
NPHASE = 4
HALF = NCHUNK // NPHASE
NGROUP = HALF // NBUF
N_ACC = N + 8
RPT = 624
TAIL_OFF = NS * RPT
ZTAIL = N_ACC - TAIL_OFF
OTAIL = N - TAIL_OFF

_SC_MESH = plsc.VectorSubcoreMesh(core_axis_name="c", subcore_axis_name="s")


@functools.partial(
    pl.kernel,
    mesh=_SC_MESH,
    out_type=jax.ShapeDtypeStruct((NC, N, H), jnp.float32),
    scratch_types=[
        pltpu.VMEM((HALF, CH), jnp.int32),
        pltpu.VMEM((HALF, CH), jnp.int32),
        [pltpu.VMEM((CH, H), jnp.float32)] * NBUF,
        pltpu.VMEM_SHARED((N_ACC, H), jnp.float32),
        [pltpu.SemaphoreType.DMA] * NBUF,
        [pltpu.SemaphoreType.DMA] * NBUF,
    ],
)
def _sc_scatter(m_hbm, src_hbm, dst_hbm, zeros_hbm, out_hbm,
                src_v, dst_v, rows, agg_sh, sg, ss):
    c = lax.axis_index("c")
    s = lax.axis_index("s")
    wid = c * NS + s
    pltpu.sync_copy(src_hbm.at[wid, 0], src_v)
    pltpu.sync_copy(dst_hbm.at[wid, 0], dst_v)
    for b in range(NBUF):
        pltpu.async_copy(m_hbm.at[src_v.at[b]], rows[b], sg[b])
    pltpu.sync_copy(zeros_hbm.at[pl.ds(s * RPT, RPT)],
                    agg_sh.at[pl.ds(s * RPT, RPT)])

    @pl.when(s == NS - 1)
    def _zero_tail():
        pltpu.sync_copy(zeros_hbm.at[pl.ds(TAIL_OFF, ZTAIL)],
                        agg_sh.at[pl.ds(TAIL_OFF, ZTAIL)])
    plsc.subcore_barrier()

    for ph in range(NPHASE):
        if ph > 0:
            pltpu.sync_copy(src_hbm.at[wid, ph], src_v)
            pltpu.sync_copy(dst_hbm.at[wid, ph], dst_v)
            for b in range(NBUF):
                pltpu.async_copy(m_hbm.at[src_v.at[b]], rows[b], sg[b])

        def group(g, carry):
            base = g * NBUF
            for b in range(NBUF):
                j = base + b
                pltpu.make_async_copy(m_hbm.at[src_v.at[j]], rows[b],
                                      sg[b]).wait()
                pltpu.async_copy(rows[b], agg_sh.at[dst_v.at[j]], ss[b],
                                 add=True)

            @pl.when(g < NGROUP - 1)
            def _prefetch():
                for b in range(NBUF):
                    j = base + b
                    pltpu.make_async_copy(rows[b], agg_sh.at[dst_v.at[j]],
                                          ss[b]).wait()
                    pltpu.async_copy(m_hbm.at[src_v.at[j + NBUF]], rows[b],
                                     sg[b])
            return carry

        lax.fori_loop(0, NGROUP, group, 0)
        for b in range(NBUF):
            j = (NGROUP - 1) * NBUF + b
            pltpu.make_async_copy(rows[b], agg_sh.at[dst_v.at[j]],
                                  ss[b]).wait()
    plsc.subcore_barrier()
    pltpu.sync_copy(agg_sh.at[pl.ds(s * RPT, RPT)],
                    out_hbm.at[c, pl.ds(s * RPT, RPT)])

    @pl.when(s == NS - 1)
    def _out_tail():
        pltpu.sync_copy(agg_sh.at[pl.ds(TAIL_OFF, OTAIL)],
                        out_hbm.at[c, pl.ds(TAIL_OFF, OTAIL)])


_RB = 5000
_GRID = N // _RB


def _mm_body(x_ref, w_ref, o_ref):
    o_ref[...] = jnp.dot(x_ref[...], w_ref[...],
                         preferred_element_type=jnp.float32)


_mm = pl.pallas_call(
    _mm_body,
    grid=(_GRID,),
    in_specs=[
        pl.BlockSpec((_RB, H), lambda i: (i, 0)),
        pl.BlockSpec((H, H), lambda i: (0, 0)),
    ],
    out_specs=pl.BlockSpec((_RB, H), lambda i: (i, 0)),
    out_shape=jax.ShapeDtypeStruct((N, H), jnp.float32),
)


def _gru_math(parts_ref, x_ref, wih_ref, whh_ref, bih_ref, bhh_ref):
    agg = parts_ref[0] + parts_ref[1]
    x = x_ref[...]
    gi = jnp.dot(agg, wih_ref[...], preferred_element_type=jnp.float32) \
        + bih_ref[...]
    gh = jnp.dot(x, whh_ref[...], preferred_element_type=jnp.float32) \
        + bhh_ref[...]
    r = jax.nn.sigmoid(gi[:, :H] + gh[:, :H])
    zg = jax.nn.sigmoid(gi[:, H:2 * H] + gh[:, H:2 * H])
    n = jnp.tanh(gi[:, 2 * H:] + r * gh[:, 2 * H:])
    return (1.0 - zg) * n + zg * x


def _gru_body(parts_ref, x_ref, wn_ref, wih_ref, whh_ref, bih_ref, bhh_ref,
              xo_ref, mo_ref):
    xn = _gru_math(parts_ref, x_ref, wih_ref, whh_ref, bih_ref, bhh_ref)
    xo_ref[...] = xn
    mo_ref[...] = jnp.dot(xn, wn_ref[...], preferred_element_type=jnp.float32)


def _gru_last_body(parts_ref, x_ref, wih_ref, whh_ref, bih_ref, bhh_ref,
                   xo_ref):
    xo_ref[...] = _gru_math(parts_ref, x_ref, wih_ref, whh_ref, bih_ref,
                            bhh_ref)


_GRU_IN_SPECS = [
    pl.BlockSpec((NC, _RB, H), lambda i: (0, i, 0)),
    pl.BlockSpec((_RB, H), lambda i: (i, 0)),
]
_W_SPECS = [
    pl.BlockSpec((H, 3 * H), lambda i: (0, 0)),
    pl.BlockSpec((H, 3 * H), lambda i: (0, 0)),
    pl.BlockSpec((1, 3 * H), lambda i: (0, 0)),
    pl.BlockSpec((1, 3 * H), lambda i: (0, 0)),
]

_gru = pl.pallas_call(
    _gru_body,
    grid=(_GRID,),
    in_specs=_GRU_IN_SPECS + [pl.BlockSpec((H, H), lambda i: (0, 0))]
    + _W_SPECS,
    out_specs=[
        pl.BlockSpec((_RB, H), lambda i: (i, 0)),
        pl.BlockSpec((_RB, H), lambda i: (i, 0)),
    ],
    out_shape=[
        jax.ShapeDtypeStruct((N, H), jnp.float32),
        jax.ShapeDtypeStruct((N, H), jnp.float32),
    ],
)

_gru_last = pl.pallas_call(
    _gru_last_body,
    grid=(_GRID,),
    in_specs=_GRU_IN_SPECS + _W_SPECS,
    out_specs=pl.BlockSpec((_RB, H), lambda i: (i, 0)),
    out_shape=jax.ShapeDtypeStruct((N, H), jnp.float32),
)


def kernel(z, edge_index, weight, W_ih, W_hh, b_ih, b_hh):
    pad = E_PAD - E
    pad_src = (jnp.arange(pad, dtype=jnp.int32) * 127) % N
    pad_dst = N + (jnp.arange(pad, dtype=jnp.int32) % (N_ACC - N))
    src = jnp.concatenate(
        [edge_index[0].astype(jnp.int32), pad_src]).reshape(
            NW, NPHASE, HALF, CH)
    dst = jnp.concatenate(
        [edge_index[1].astype(jnp.int32), pad_dst]).reshape(
            NW, NPHASE, HALF, CH)
    W_ihT = W_ih.T.astype(jnp.float32)
    W_hhT = W_hh.T.astype(jnp.float32)
    b_ih2 = b_ih.reshape(1, 3 * H)
    b_hh2 = b_hh.reshape(1, 3 * H)
    zeros = jnp.zeros((N_ACC, H), jnp.float32)

    x = z
    m = _mm(x, weight[0])
    for i in range(LAYERS):
        parts = _sc_scatter(m, src, dst, zeros)
        if i < LAYERS - 1:
            x, m = _gru(parts, x, weight[i + 1], W_ihT, W_hhT, b_ih2, b_hh2)
        else:
            x = _gru_last(parts, x, W_ihT, W_hhT, b_ih2, b_hh2)
    return x
